# baseline (speedup 1.0000x reference)
import functools

import jax
import jax.numpy as jnp
from jax import lax
from jax.experimental import pallas as pl
from jax.experimental.pallas import tpu as pltpu
from jax.experimental.pallas import tpu_sc as plsc

N_NODES = 10000
E_EDGES = 160000
D_FEAT = 256
D_EDGE = 16
HIDDEN = 256
OUT = 256

NC = 2
NS = 16
LANES = 16
G = E_EDGES // 128
GPC = G // NC
NB = 25
NCH = GPC // NB
N_PAD = 10240


def _sc_segment_sum(ea_v, idx_v):
    mesh = plsc.VectorSubcoreMesh(
        core_axis_name="c", subcore_axis_name="s", num_cores=NC, num_subcores=NS
    )

    @functools.partial(
        pl.kernel,
        out_type=jax.ShapeDtypeStruct((NC, 2, 8, N_PAD), jnp.float32),
        name="sc_segment_sum",
        mesh=mesh,
        compiler_params=pltpu.CompilerParams(
            use_tc_tiling_on_sc=False, needs_layout_passes=False
        ),
        scratch_types=[
            pltpu.VMEM((2, NB, 128), jnp.float32),
            pltpu.VMEM((2, NB, 128), jnp.int32),
            pltpu.VMEM((N_PAD,), jnp.float32),
            pltpu.SemaphoreType.DMA,
            pltpu.SemaphoreType.DMA,
        ],
    )
    def k(ea_hbm, idx_hbm, out_hbm, val_v, ind_v, acc_v, sem0, sem1):
        cid = lax.axis_index("c")
        sid = lax.axis_index("s")
        fa = sid // 8
        fr = sid % 8
        gbase = cid * GPC
        sems = (sem0, sem1)

        def start(ch, buf):
            g0 = gbase + ch * NB
            dv = pltpu.async_copy(
                ea_hbm.at[fa, pl.ds(g0, NB), fr], val_v.at[buf], sems[buf]
            )
            di = pltpu.async_copy(
                idx_hbm.at[pl.ds(g0, NB), 0], ind_v.at[buf], sems[buf]
            )
            return dv, di

        pend = start(0, 0)

        @plsc.parallel_loop(0, N_PAD, step=LANES)
        def _(i):
            acc_v[pl.ds(i, LANES)] = jnp.zeros((LANES,), jnp.float32)

        for ch in range(NCH):
            buf = ch % 2
            nxt = start(ch + 1, 1 - buf) if ch + 1 < NCH else None
            pend[0].wait()
            pend[1].wait()

            @plsc.parallel_loop(0, NB, step=1, unroll=2)
            def _(j):
                for u in range(8):
                    v = val_v[buf, j, pl.ds(u * LANES, LANES)]
                    ix = ind_v[buf, j, pl.ds(u * LANES, LANES)]
                    plsc.addupdate_scatter(acc_v, [ix], v)

            pend = nxt

        pltpu.sync_copy(acc_v, out_hbm.at[cid, fa, fr])

    return k(ea_v, idx_v)


def _tc_mlp(x, partials, w1x, w1m, b1, w2, b2):
    tiles = 8
    cols = tiles * 128
    grid = (N_PAD // cols,)

    def body(x_ref, p_ref, w1x_ref, w1m_ref, b1_ref, w2_ref, b2_ref, o_ref):
        psum = p_ref[0] + p_ref[1]
        h = jnp.dot(
            x_ref[...].astype(jnp.bfloat16),
            w1x_ref[...].astype(jnp.bfloat16),
            preferred_element_type=jnp.float32,
        )
        hm = []
        for t in range(tiles):
            msg_t = jnp.concatenate([psum[0, :, t], psum[1, :, t]], axis=0)
            hm.append(
                lax.dot_general(
                    msg_t, w1m_ref[...], (((0,), (0,)), ((), ())),
                    preferred_element_type=jnp.float32,
                )
            )
        h = h + jnp.concatenate(hm, axis=0)
        h = jnp.maximum(h + b1_ref[...], 0.0)
        o_ref[...] = (
            jnp.dot(
                h.astype(jnp.bfloat16),
                w2_ref[...].astype(jnp.bfloat16),
                preferred_element_type=jnp.float32,
            )
            + b2_ref[...]
        )

    return pl.pallas_call(
        body,
        grid=grid,
        in_specs=[
            pl.BlockSpec((cols, D_FEAT), lambda i: (i, 0)),
            pl.BlockSpec((NC, 2, 8, tiles, 128), lambda i: (0, 0, 0, i, 0)),
            pl.BlockSpec((D_FEAT, HIDDEN), lambda i: (0, 0)),
            pl.BlockSpec((D_EDGE, HIDDEN), lambda i: (0, 0)),
            pl.BlockSpec((1, HIDDEN), lambda i: (0, 0)),
            pl.BlockSpec((HIDDEN, OUT), lambda i: (0, 0)),
            pl.BlockSpec((1, OUT), lambda i: (0, 0)),
        ],
        out_specs=pl.BlockSpec((cols, OUT), lambda i: (i, 0)),
        out_shape=jax.ShapeDtypeStruct((N_NODES, OUT), jnp.float32),
    )(x, partials, w1x, w1m, b1.reshape(1, HIDDEN), w2, b2.reshape(1, OUT))


def kernel(x, edge_index, edge_attr, W1, b1, W2, b2):
    ea_v = edge_attr.T.reshape(2, 8, 1250, 128).transpose(0, 2, 1, 3)
    idx_v = edge_index.reshape(2, 1250, 128).transpose(1, 0, 2)
    partials = _sc_segment_sum(ea_v, idx_v)
    partials = partials.reshape(NC, 2, 8, N_PAD // 128, 128)
    return _tc_mlp(x, partials, W1[:D_FEAT], W1[D_FEAT:], b1, W2, b2)


# speedup vs baseline: 1.2966x; 1.2966x over previous
import functools

import jax
import jax.numpy as jnp
from jax import lax
from jax.experimental import pallas as pl
from jax.experimental.pallas import tpu as pltpu
from jax.experimental.pallas import tpu_sc as plsc

N_NODES = 10000
E_EDGES = 160000
D_FEAT = 256
D_EDGE = 16
HIDDEN = 256
OUT = 256

NC = 2
NS = 16
LANES = 16
G = E_EDGES // 128
GPC = G // NC
NB = 125
NCH = GPC // NB
N_PAD = 10240


def _sc_segment_sum(ea_v, idx_v):
    mesh = plsc.VectorSubcoreMesh(
        core_axis_name="c", subcore_axis_name="s", num_cores=NC, num_subcores=NS
    )

    @functools.partial(
        pl.kernel,
        out_type=jax.ShapeDtypeStruct((NC, 2, 8, N_PAD), jnp.float32),
        name="sc_segment_sum",
        mesh=mesh,
        compiler_params=pltpu.CompilerParams(
            use_tc_tiling_on_sc=False, needs_layout_passes=False
        ),
        scratch_types=[
            pltpu.VMEM((2, NB, 128), jnp.float32),
            pltpu.VMEM((2, NB, 128), jnp.int32),
            pltpu.VMEM((N_PAD,), jnp.float32),
            pltpu.SemaphoreType.DMA,
            pltpu.SemaphoreType.DMA,
        ],
    )
    def k(ea_hbm, idx_hbm, out_hbm, val_v, ind_v, acc_v, sem0, sem1):
        cid = lax.axis_index("c")
        sid = lax.axis_index("s")
        fa = sid // 8
        fr = sid % 8
        gbase = cid * GPC
        sems = (sem0, sem1)

        def start(ch, buf):
            g0 = gbase + ch * NB
            dv = pltpu.async_copy(
                ea_hbm.at[fa, pl.ds(g0, NB), fr], val_v.at[buf], sems[buf]
            )
            di = pltpu.async_copy(
                idx_hbm.at[pl.ds(g0, NB), 0], ind_v.at[buf], sems[buf]
            )
            return dv, di

        pend = start(0, 0)

        @plsc.parallel_loop(0, N_PAD, step=LANES)
        def _(i):
            acc_v[pl.ds(i, LANES)] = jnp.zeros((LANES,), jnp.float32)

        for ch in range(NCH):
            buf = ch % 2
            nxt = start(ch + 1, 1 - buf) if ch + 1 < NCH else None
            pend[0].wait()
            pend[1].wait()

            @plsc.parallel_loop(0, NB, step=1, unroll=4)
            def _(j):
                for u in range(8):
                    v = val_v[buf, j, pl.ds(u * LANES, LANES)]
                    ix = ind_v[buf, j, pl.ds(u * LANES, LANES)]
                    plsc.addupdate_scatter(acc_v, [ix], v)

            pend = nxt

        pltpu.sync_copy(acc_v, out_hbm.at[cid, fa, fr])

    return k(ea_v, idx_v)


def _tc_mlp(x, partials, w1x, w1m, b1, w2, b2):
    tiles = 16
    cols = tiles * 128
    grid = (N_PAD // cols,)

    def body(x_ref, p_ref, w1x_ref, w1m_ref, b1_ref, w2_ref, b2_ref, o_ref):
        psum = p_ref[0] + p_ref[1]
        h = jnp.dot(
            x_ref[...].astype(jnp.bfloat16),
            w1x_ref[...].astype(jnp.bfloat16),
            preferred_element_type=jnp.float32,
        )
        hm = []
        for t in range(tiles):
            msg_t = jnp.concatenate([psum[0, :, t], psum[1, :, t]], axis=0)
            hm.append(
                lax.dot_general(
                    msg_t, w1m_ref[...], (((0,), (0,)), ((), ())),
                    preferred_element_type=jnp.float32,
                )
            )
        h = h + jnp.concatenate(hm, axis=0)
        h = jnp.maximum(h + b1_ref[...], 0.0)
        o_ref[...] = (
            jnp.dot(
                h.astype(jnp.bfloat16),
                w2_ref[...].astype(jnp.bfloat16),
                preferred_element_type=jnp.float32,
            )
            + b2_ref[...]
        )

    return pl.pallas_call(
        body,
        grid=grid,
        in_specs=[
            pl.BlockSpec((cols, D_FEAT), lambda i: (i, 0)),
            pl.BlockSpec((NC, 2, 8, tiles, 128), lambda i: (0, 0, 0, i, 0)),
            pl.BlockSpec((D_FEAT, HIDDEN), lambda i: (0, 0)),
            pl.BlockSpec((D_EDGE, HIDDEN), lambda i: (0, 0)),
            pl.BlockSpec((1, HIDDEN), lambda i: (0, 0)),
            pl.BlockSpec((HIDDEN, OUT), lambda i: (0, 0)),
            pl.BlockSpec((1, OUT), lambda i: (0, 0)),
        ],
        out_specs=pl.BlockSpec((cols, OUT), lambda i: (i, 0)),
        out_shape=jax.ShapeDtypeStruct((N_NODES, OUT), jnp.float32),
    )(x, partials, w1x, w1m, b1.reshape(1, HIDDEN), w2, b2.reshape(1, OUT))


def kernel(x, edge_index, edge_attr, W1, b1, W2, b2):
    ea_v = edge_attr.T.reshape(2, 8, 1250, 128).transpose(0, 2, 1, 3)
    idx_v = edge_index.reshape(2, 1250, 128).transpose(1, 0, 2)
    partials = _sc_segment_sum(ea_v, idx_v)
    partials = partials.reshape(NC, 2, 8, N_PAD // 128, 128)
    return _tc_mlp(x, partials, W1[:D_FEAT], W1[D_FEAT:], b1, W2, b2)


# speedup vs baseline: 1.3475x; 1.0393x over previous
import functools

import jax
import jax.numpy as jnp
from jax import lax
from jax.experimental import pallas as pl
from jax.experimental.pallas import tpu as pltpu
from jax.experimental.pallas import tpu_sc as plsc

N_NODES = 10000
E_EDGES = 160000
D_FEAT = 256
D_EDGE = 16
HIDDEN = 256
OUT = 256

NC = 2
NS = 16
LANES = 16
G = E_EDGES // 128
GPC = G // NC
HALF0 = 313
NB = 64
NCH = 5
N_PAD = 10240


def _sc_segment_sum(ea_v, idx_v):
    mesh = plsc.VectorSubcoreMesh(
        core_axis_name="c", subcore_axis_name="s", num_cores=NC, num_subcores=NS
    )

    @functools.partial(
        pl.kernel,
        out_type=jax.ShapeDtypeStruct((NC, 2, 8, N_PAD), jnp.float32),
        name="sc_segment_sum",
        mesh=mesh,
        compiler_params=pltpu.CompilerParams(
            use_tc_tiling_on_sc=False, needs_layout_passes=False
        ),
        scratch_types=[
            pltpu.VMEM((2, NB, 2, 128), jnp.float32),
            pltpu.VMEM((2, NB, 128), jnp.int32),
            pltpu.VMEM((N_PAD,), jnp.float32),
            pltpu.VMEM((N_PAD,), jnp.float32),
            pltpu.SemaphoreType.DMA,
            pltpu.SemaphoreType.DMA,
        ],
    )
    def k(ea_hbm, idx_hbm, out_hbm, val_v, ind_v, acc0_v, acc1_v, sem0, sem1):
        cid = lax.axis_index("c")
        sid = lax.axis_index("s")
        p = sid % 8
        h = sid // 8
        fa = p // 4
        fr = 2 * (p % 4)
        size_h = HALF0 - h
        hbase = cid * GPC + h * HALF0
        sems = (sem0, sem1)

        def start(ch, buf):
            c0 = jnp.minimum(ch * NB, size_h - NB)
            g0 = hbase + c0
            dv = pltpu.async_copy(
                ea_hbm.at[fa, pl.ds(g0, NB), pl.ds(fr, 2)], val_v.at[buf], sems[buf]
            )
            di = pltpu.async_copy(
                idx_hbm.at[pl.ds(g0, NB), 0], ind_v.at[buf], sems[buf]
            )
            return dv, di

        pend = start(0, 0)

        @plsc.parallel_loop(0, N_PAD, step=LANES)
        def _(i):
            acc0_v[pl.ds(i, LANES)] = jnp.zeros((LANES,), jnp.float32)
            acc1_v[pl.ds(i, LANES)] = jnp.zeros((LANES,), jnp.float32)

        for ch in range(NCH):
            buf = ch % 2
            nxt = start(ch + 1, 1 - buf) if ch + 1 < NCH else None
            pend[0].wait()
            pend[1].wait()
            j0 = jnp.maximum(ch * NB - (size_h - NB), 0)

            @plsc.parallel_loop(j0, NB, step=1, unroll=4)
            def _(j):
                for u in range(8):
                    ix = ind_v[buf, j, pl.ds(u * LANES, LANES)]
                    v0 = val_v[buf, j, 0, pl.ds(u * LANES, LANES)]
                    v1 = val_v[buf, j, 1, pl.ds(u * LANES, LANES)]
                    plsc.addupdate_scatter(acc0_v, [ix], v0)
                    plsc.addupdate_scatter(acc1_v, [ix], v1)

            pend = nxt

        pltpu.sync_copy(acc0_v, out_hbm.at[cid, fa, fr])
        pltpu.sync_copy(acc1_v, out_hbm.at[cid, fa, fr + 1])

    return k(ea_v, idx_v)


def _tc_mlp(x, partials, w1x, w1m, b1, w2, b2):
    tiles = 16
    cols = tiles * 128
    grid = (N_PAD // cols,)

    def body(x_ref, p_ref, w1x_ref, w1m_ref, b1_ref, w2_ref, b2_ref, o_ref):
        psum = p_ref[0] + p_ref[1]
        h = jnp.dot(
            x_ref[...].astype(jnp.bfloat16),
            w1x_ref[...].astype(jnp.bfloat16),
            preferred_element_type=jnp.float32,
        )
        hm = []
        for t in range(tiles):
            msg_t = jnp.concatenate([psum[0, :, t], psum[1, :, t]], axis=0)
            hm.append(
                lax.dot_general(
                    msg_t, w1m_ref[...], (((0,), (0,)), ((), ())),
                    preferred_element_type=jnp.float32,
                )
            )
        h = h + jnp.concatenate(hm, axis=0)
        h = jnp.maximum(h + b1_ref[...], 0.0)
        o_ref[...] = (
            jnp.dot(
                h.astype(jnp.bfloat16),
                w2_ref[...].astype(jnp.bfloat16),
                preferred_element_type=jnp.float32,
            )
            + b2_ref[...]
        )

    return pl.pallas_call(
        body,
        grid=grid,
        in_specs=[
            pl.BlockSpec((cols, D_FEAT), lambda i: (i, 0)),
            pl.BlockSpec((NC, 2, 8, tiles, 128), lambda i: (0, 0, 0, i, 0)),
            pl.BlockSpec((D_FEAT, HIDDEN), lambda i: (0, 0)),
            pl.BlockSpec((D_EDGE, HIDDEN), lambda i: (0, 0)),
            pl.BlockSpec((1, HIDDEN), lambda i: (0, 0)),
            pl.BlockSpec((HIDDEN, OUT), lambda i: (0, 0)),
            pl.BlockSpec((1, OUT), lambda i: (0, 0)),
        ],
        out_specs=pl.BlockSpec((cols, OUT), lambda i: (i, 0)),
        out_shape=jax.ShapeDtypeStruct((N_NODES, OUT), jnp.float32),
    )(x, partials, w1x, w1m, b1.reshape(1, HIDDEN), w2, b2.reshape(1, OUT))


def kernel(x, edge_index, edge_attr, W1, b1, W2, b2):
    ea_v = edge_attr.T.reshape(2, 8, 1250, 128).transpose(0, 2, 1, 3)
    idx_v = edge_index.reshape(2, 1250, 128).transpose(1, 0, 2)
    partials = _sc_segment_sum(ea_v, idx_v)
    partials = partials.reshape(NC, 2, 8, N_PAD // 128, 128)
    return _tc_mlp(x, partials, W1[:D_FEAT], W1[D_FEAT:], b1, W2, b2)
